# baseline (speedup 1.0000x reference)
import functools

import jax
import jax.numpy as jnp
from jax import lax
from jax.experimental import pallas as pl
from jax.experimental.pallas import tpu as pltpu
from jax.experimental.pallas import tpu_sc as plsc

_NUM_CORES = 2
_NUM_SUBCORES = 16
_NUM_WORKERS = _NUM_CORES * _NUM_SUBCORES

_CHUNK = 64


def _embed_kernel(b_per_w, n_chunks, hidden, table_hbm, labels_hbm, out_hbm,
                  idx_v, rows_v, sem):
  wid = lax.axis_index("s") * _NUM_CORES + lax.axis_index("c")
  base = wid * b_per_w
  pltpu.sync_copy(labels_hbm.at[pl.ds(base, b_per_w)], idx_v)
  for c in range(n_chunks):
    pltpu.async_copy(
        table_hbm.at[idx_v.at[pl.ds(c * _CHUNK, _CHUNK)]], rows_v, sem
    ).wait()
    pltpu.sync_copy(rows_v, out_hbm.at[pl.ds(base + c * _CHUNK, _CHUNK)])


def kernel(labels, table):
  batch = labels.shape[0]
  hidden = table.shape[1]
  b_per_w = batch // _NUM_WORKERS
  n_chunks = b_per_w // _CHUNK

  mesh = plsc.VectorSubcoreMesh(
      core_axis_name="c", subcore_axis_name="s",
      num_cores=_NUM_CORES, num_subcores=_NUM_SUBCORES)

  run = pl.kernel(
      functools.partial(_embed_kernel, b_per_w, n_chunks, hidden),
      out_type=jax.ShapeDtypeStruct((batch, hidden), jnp.float32),
      mesh=mesh,
      scratch_types=[
          pltpu.VMEM((b_per_w,), jnp.int32),
          pltpu.VMEM((_CHUNK, hidden), jnp.float32),
          pltpu.SemaphoreType.DMA,
      ],
  )
  out = run(table, labels.astype(jnp.int32))
  return out[:, None, :]
